# baseline (speedup 1.0000x reference)
import functools

import jax
import jax.numpy as jnp
from jax import lax
from jax.experimental import pallas as pl
from jax.experimental.pallas import tpu as pltpu
from jax.experimental.pallas import tpu_sc as plsc

B = 16384
NF = 26
VOCAB = 100000
H = 64

NC = 2
NS = 16
NW = NC * NS

ROWS_PER_TILE = B // NW
CHUNK_ROWS = 4
CHUNK_IDX = CHUNK_ROWS * NF
NCHUNK = ROWS_PER_TILE // CHUNK_ROWS
IDX_PER_TILE = ROWS_PER_TILE * NF
H_WORDS = ROWS_PER_TILE * H


def _sc_gather_sum_build():
    mesh = plsc.VectorSubcoreMesh(core_axis_name="c", subcore_axis_name="s")

    @functools.partial(
        pl.kernel,
        out_type=jax.ShapeDtypeStruct((B * H,), jnp.float32),
        mesh=mesh,
        compiler_params=pltpu.CompilerParams(use_tc_tiling_on_sc=False),
        scratch_types=[
            pltpu.VMEM((IDX_PER_TILE,), jnp.int32),
            pltpu.VMEM((CHUNK_IDX, H), jnp.float32),
            pltpu.VMEM((CHUNK_IDX, H), jnp.float32),
            pltpu.VMEM((H_WORDS,), jnp.float32),
            pltpu.SemaphoreType.DMA,
            pltpu.SemaphoreType.DMA,
        ],
    )
    def sc_gather_sum(table_hbm, idx_hbm, out_hbm, idx_v, buf0, buf1, h_v,
                      sem0, sem1):
        wid = lax.axis_index("s") * NC + lax.axis_index("c")

        pltpu.sync_copy(idx_hbm.at[pl.ds(wid * IDX_PER_TILE, IDX_PER_TILE)],
                        idx_v)

        def start(g, buf, sem):
            sl = idx_v.at[pl.ds(pl.multiple_of(g * CHUNK_IDX, 8), CHUNK_IDX)]
            pltpu.async_copy(table_hbm.at[sl], buf, sem)

        def wait(buf, sem):
            pltpu.make_async_copy(table_hbm.at[pl.ds(0, CHUNK_IDX)], buf,
                                  sem).wait()

        def accumulate(g, buf):
            for r in range(CHUNK_ROWS):
                for c in range(H // 16):
                    acc = buf[r * NF, pl.ds(c * 16, 16)]
                    for j in range(1, NF):
                        acc = acc + buf[r * NF + j, pl.ds(c * 16, 16)]
                    h_v[pl.ds((g * CHUNK_ROWS + r) * H + c * 16, 16)] = acc

        start(0, buf0, sem0)
        start(1, buf1, sem1)

        def body(t, carry):
            g = t * 2
            wait(buf0, sem0)
            accumulate(g, buf0)
            start(g + 2, buf0, sem0)
            wait(buf1, sem1)
            accumulate(g + 1, buf1)
            start(g + 3, buf1, sem1)
            return carry

        lax.fori_loop(0, NCHUNK // 2 - 1, body, 0)

        wait(buf0, sem0)
        accumulate(NCHUNK - 2, buf0)
        wait(buf1, sem1)
        accumulate(NCHUNK - 1, buf1)

        pltpu.sync_copy(h_v, out_hbm.at[pl.ds(wid * H_WORDS, H_WORDS)])

    return sc_gather_sum


_sc_gather_sum = _sc_gather_sum_build()

_TC_BLK = 2048


def _tc_linear_body(h_ref, w_ref, b_ref, o_ref):
    o_ref[...] = jnp.maximum(
        jnp.dot(h_ref[...], w_ref[...], preferred_element_type=jnp.float32)
        + b_ref[...],
        0.0,
    )


def _tc_linear(h, w_t, b2d):
    return pl.pallas_call(
        _tc_linear_body,
        grid=(B // _TC_BLK,),
        in_specs=[
            pl.BlockSpec((_TC_BLK, H), lambda i: (i, 0)),
            pl.BlockSpec((H, H), lambda i: (0, 0)),
            pl.BlockSpec((1, H), lambda i: (0, 0)),
        ],
        out_specs=pl.BlockSpec((_TC_BLK, H), lambda i: (i, 0)),
        out_shape=jax.ShapeDtypeStruct((B, H), jnp.float32),
    )(h, w_t, b2d)


def kernel(x, tables, fc_w, fc_b):
    offs = (jnp.arange(NF, dtype=jnp.int32) * VOCAB)[None, :]
    idx = (x.astype(jnp.int32) + offs).reshape(-1)
    table_flat = tables.reshape(NF * VOCAB, H)
    h = _sc_gather_sum(table_flat, idx).reshape(B, H)
    return _tc_linear(h, fc_w.T, fc_b.reshape(1, H))


# speedup vs baseline: 1.0594x; 1.0594x over previous
import functools

import jax
import jax.numpy as jnp
from jax import lax
from jax.experimental import pallas as pl
from jax.experimental.pallas import tpu as pltpu
from jax.experimental.pallas import tpu_sc as plsc

B = 16384
NF = 26
VOCAB = 100000
H = 64

NC = 2
NS = 16
NW = NC * NS

CH = B // NW
H_WORDS = CH * H


def _sc_gather_sum_build():
    mesh = plsc.VectorSubcoreMesh(core_axis_name="c", subcore_axis_name="s")

    @functools.partial(
        pl.kernel,
        out_type=jax.ShapeDtypeStruct((B * H,), jnp.float32),
        mesh=mesh,
        compiler_params=pltpu.CompilerParams(use_tc_tiling_on_sc=False),
        scratch_types=[
            pltpu.VMEM((NF, CH), jnp.int32),
            pltpu.VMEM((CH, H), jnp.float32),
            pltpu.VMEM((CH, H), jnp.float32),
            pltpu.VMEM((H_WORDS,), jnp.float32),
            pltpu.SemaphoreType.DMA,
            pltpu.SemaphoreType.DMA,
        ],
    )
    def sc_gather_sum(table_hbm, idx_hbm, out_hbm, idx_v, buf0, buf1, h_v,
                      sem0, sem1):
        wid = lax.axis_index("s") * NC + lax.axis_index("c")
        base = wid * CH

        pltpu.sync_copy(idx_hbm.at[:, pl.ds(base, CH)], idx_v)

        def start(f, buf, sem):
            pltpu.async_copy(table_hbm.at[f].at[idx_v.at[f]], buf, sem)

        def wait(buf, sem):
            pltpu.make_async_copy(table_hbm.at[0, pl.ds(0, CH)], buf,
                                  sem).wait()

        zeros = jnp.zeros((16,), jnp.float32)

        def zero_body(i, carry):
            for k in range(8):
                h_v[pl.ds(i * 128 + k * 16, 16)] = zeros
            return carry

        lax.fori_loop(0, H_WORDS // 128, zero_body, 0)

        def accumulate(buf):
            def acc_body(i, carry):
                for k in range(8):
                    r = i * 8 + k
                    for c in range(H // 16):
                        plsc.addupdate(h_v.at[pl.ds(r * H + c * 16, 16)],
                                       buf[r, pl.ds(c * 16, 16)])
                return carry
            lax.fori_loop(0, CH // 8, acc_body, 0)

        start(0, buf0, sem0)
        start(1, buf1, sem1)

        def body(t, carry):
            f = t * 2
            wait(buf0, sem0)
            accumulate(buf0)
            start(f + 2, buf0, sem0)
            wait(buf1, sem1)
            accumulate(buf1)
            start(f + 3, buf1, sem1)
            return carry

        lax.fori_loop(0, NF // 2 - 1, body, 0)

        wait(buf0, sem0)
        accumulate(buf0)
        wait(buf1, sem1)
        accumulate(buf1)

        pltpu.sync_copy(h_v, out_hbm.at[pl.ds(wid * H_WORDS, H_WORDS)])

    return sc_gather_sum


_sc_gather_sum = _sc_gather_sum_build()

_TC_BLK = 2048


def _tc_linear_body(h_ref, w_ref, b_ref, o_ref):
    o_ref[...] = jnp.maximum(
        jnp.dot(h_ref[...], w_ref[...], preferred_element_type=jnp.float32)
        + b_ref[...],
        0.0,
    )


def _tc_linear(h, w_t, b2d):
    return pl.pallas_call(
        _tc_linear_body,
        grid=(B // _TC_BLK,),
        in_specs=[
            pl.BlockSpec((_TC_BLK, H), lambda i: (i, 0)),
            pl.BlockSpec((H, H), lambda i: (0, 0)),
            pl.BlockSpec((1, H), lambda i: (0, 0)),
        ],
        out_specs=pl.BlockSpec((_TC_BLK, H), lambda i: (i, 0)),
        out_shape=jax.ShapeDtypeStruct((B, H), jnp.float32),
    )(h, w_t, b2d)


def kernel(x, tables, fc_w, fc_b):
    idx_t = jnp.transpose(x.astype(jnp.int32))
    h = _sc_gather_sum(tables, idx_t).reshape(B, H)
    return _tc_linear(h, fc_w.T, fc_b.reshape(1, H))
